# baseline (speedup 1.0000x reference)
import functools

import jax
import jax.numpy as jnp
from jax import lax
from jax.experimental import pallas as pl
from jax.experimental.pallas import tpu as pltpu
from jax.experimental.pallas import tpu_sc as plsc

_NPOINTS = 512
_LANES = 128
_CH = 32

_SC_B = 4
_KC = 2
_TILE_PTS = 4096
_UNROLL = 4


def _amax_lanes(v):
    for s in (1, 2, 4, 8, 16, 32, 64):
        v = jnp.maximum(v, pltpu.roll(v, s, axis=1))
    return v


def _asum_lanes(v):
    for s in (1, 2, 4, 8, 16, 32, 64):
        v = v + pltpu.roll(v, s, axis=1)
    return v


def _fps_tc(xyz, far0, npoints):
    B, N, _ = xyz.shape
    rows = N // _LANES
    ch = min(_CH, rows)
    nchunks = rows // ch
    chunk_elems = ch * _LANES
    xyzt = jnp.transpose(xyz, (2, 0, 1)).reshape(3, B, rows, _LANES)
    xyz3 = jnp.transpose(xyz.reshape(B, rows, _LANES, 3), (0, 1, 3, 2))

    def body(far0_ref, xyzt_ref, xyz3_ref, out_ref, dist_ref):
        dist_ref[...] = jnp.full((B, rows, _LANES), 1e10, jnp.float32)
        lane3 = jax.lax.broadcasted_iota(jnp.int32, (3, _LANES), 1)
        b_iota = jax.lax.broadcasted_iota(jnp.int32, (1, B), 1)
        pos_iota = (
            jax.lax.broadcasted_iota(jnp.int32, (ch, _LANES), 0) * _LANES
            + jax.lax.broadcasted_iota(jnp.int32, (ch, _LANES), 1)
        )

        def gather_vec(b, f):
            g = xyz3_ref[b, pl.ds(f // _LANES, 1)].reshape(3, _LANES)
            return _asum_lanes(jnp.where(lane3 == f % _LANES, g, 0.0))

        def iter_body(i, carry):
            new_f = []
            new_c = []
            for b in range(B):
                cv = carry[b]
                cxb = jnp.broadcast_to(cv[0:1, :], (ch, _LANES))
                cyb = jnp.broadcast_to(cv[1:2, :], (ch, _LANES))
                czb = jnp.broadcast_to(cv[2:3, :], (ch, _LANES))
                rm = jnp.full((ch, _LANES), -1.0, jnp.float32)
                ri = jnp.zeros((ch, _LANES), jnp.int32)
                for k in range(nchunks):
                    sl = pl.ds(k * ch, ch)
                    x = xyzt_ref[0, b, sl, :]
                    y = xyzt_ref[1, b, sl, :]
                    z = xyzt_ref[2, b, sl, :]
                    dx = x - cxb
                    dy = y - cyb
                    dz = z - czb
                    d = dx * dx + dy * dy + dz * dz
                    nd = jnp.minimum(dist_ref[b, sl, :], d)
                    dist_ref[b, sl, :] = nd
                    gt = nd > rm
                    rm = jnp.where(gt, nd, rm)
                    ri = jnp.where(gt, k, ri)
                mvec = _amax_lanes(jnp.max(rm, axis=0, keepdims=True))
                f_sc = jnp.min(jnp.where(rm == mvec, ri * chunk_elems + pos_iota, N))
                new_f.append(f_sc)
                new_c.append(gather_vec(b, f_sc))
            rec = jnp.zeros((1, B), jnp.int32)
            for b in range(B):
                rec = jnp.where(b_iota == b, new_f[b], rec)
            out_ref[pl.ds(i + 1, 1), :] = rec
            return tuple(new_c)

        rec = jnp.zeros((1, B), jnp.int32)
        carry0 = []
        for b in range(B):
            f0 = far0_ref[b]
            rec = jnp.where(b_iota == b, f0, rec)
            carry0.append(gather_vec(b, f0))
        out_ref[pl.ds(0, 1), :] = rec

        jax.lax.fori_loop(0, npoints, iter_body, tuple(carry0))

    out = pl.pallas_call(
        body,
        grid=(),
        in_specs=[
            pl.BlockSpec(memory_space=pltpu.SMEM),
            pl.BlockSpec(memory_space=pltpu.VMEM),
            pl.BlockSpec(memory_space=pltpu.VMEM),
        ],
        out_specs=pl.BlockSpec(memory_space=pltpu.VMEM),
        out_shape=jax.ShapeDtypeStruct((npoints + 1, B), jnp.int32),
        scratch_shapes=[pltpu.VMEM((B, rows, _LANES), jnp.float32)],
    )(far0, xyzt, xyz3)
    return out[:npoints].T


def _fps_sc(xyzsc, far0sc, npoints):
    nchunks = _TILE_PTS // 16
    mesh = plsc.VectorSubcoreMesh(core_axis_name="c", subcore_axis_name="s")

    @functools.partial(
        pl.kernel,
        mesh=mesh,
        out_type=jax.ShapeDtypeStruct((2, npoints, 128), jnp.float32),
        scratch_types=[
            pltpu.VMEM((3, _KC, _TILE_PTS), jnp.float32),
            pltpu.VMEM((_KC, _TILE_PTS), jnp.float32),
            pltpu.VMEM((128,), jnp.int32),
            pltpu.VMEM((128,), jnp.float32),
            pltpu.VMEM((16, 128), jnp.float32),
            pltpu.VMEM((128,), jnp.float32),
            pltpu.VMEM((_KC, 128), jnp.float32),
            pltpu.VMEM((128,), jnp.float32),
            pltpu.VMEM_SHARED((16 + _KC + npoints, 128), jnp.float32),
        ],
    )
    def k(xyz_hbm, far0_hbm, out_hbm, xyz_sp, dist_sp, far_land, stage_pub,
          red_land, cent_stage, cent_land, out_stage, shared):
        c = lax.axis_index("c")
        s = lax.axis_index("s")
        lane16 = lax.iota(jnp.int32, 16)

        def bfly(v, op):
            for sh in (1, 2, 4, 8):
                p = v.at[(lane16 + sh) % 16].get(mode="promise_in_bounds")
                v = op(v, p)
            return v

        def lane_f32(v, l):
            return bfly(jnp.where(lane16 == l, v, -jnp.inf), jnp.maximum)[0]

        def lane_i32(v, l):
            return bfly(jnp.where(lane16 == l, v, 0), jnp.maximum)[0]

        pltpu.sync_copy(xyz_hbm.at[c, s], xyz_sp)
        pltpu.sync_copy(far0_hbm.at[c], far_land)
        for j in range(_KC):
            def init_d(ii, _, j=j):
                dist_sp[j, pl.ds(ii * 16, 16)] = jnp.full((16,), 1e10, jnp.float32)
                return 0
            lax.fori_loop(0, nchunks, init_d, 0)

        def publish_cent(j, f):
            owner = f // _TILE_PTS

            @pl.when(s == owner)
            def _():
                lp = f % _TILE_PTS
                base = (lp // 16) * 16
                sel = lane16 == lp - base
                cx = bfly(jnp.where(sel, xyz_sp[0, j, pl.ds(base, 16)],
                                    -jnp.inf), jnp.maximum)[0]
                cy = bfly(jnp.where(sel, xyz_sp[1, j, pl.ds(base, 16)],
                                    -jnp.inf), jnp.maximum)[0]
                cz = bfly(jnp.where(sel, xyz_sp[2, j, pl.ds(base, 16)],
                                    -jnp.inf), jnp.maximum)[0]
                crow = jnp.where(
                    lane16 == 0, cx,
                    jnp.where(lane16 == 1, cy,
                              jnp.where(lane16 == 2, cz, 0.0)))
                cent_stage[pl.ds(0, 16)] = crow
                pltpu.sync_copy(cent_stage, shared.at[16 + j])

        farv = far_land[pl.ds(0, 16)]
        for j in range(_KC):
            publish_cent(j, lane_i32(farv, j))
        plsc.subcore_barrier()
        pltpu.sync_copy(shared.at[pl.ds(16, _KC)], cent_land)

        def iter_body(i, _):
            stagev = jnp.zeros((16,), jnp.float32)
            for j in range(_KC):
                crow = cent_land[j, pl.ds(0, 16)]
                cx = crow[0]
                cy = crow[1]
                cz = crow[2]

                def chunk(ii, carry, j=j, cx=cx, cy=cy, cz=cz):
                    rm, ri = carry
                    for u in range(_UNROLL):
                        kk = ii * _UNROLL + u
                        sl = pl.ds(kk * 16, 16)
                        x = xyz_sp[0, j, sl]
                        y = xyz_sp[1, j, sl]
                        z = xyz_sp[2, j, sl]
                        dx = x - cx
                        dy = y - cy
                        dz = z - cz
                        d = dx * dx + dy * dy + dz * dz
                        nd = jnp.minimum(dist_sp[j, sl], d)
                        dist_sp[j, sl] = nd
                        gt = nd > rm
                        rm = jnp.where(gt, nd, rm)
                        ri = jnp.where(gt, jnp.full((16,), kk, jnp.int32), ri)
                    return rm, ri

                rm = jnp.full((16,), -1.0, jnp.float32)
                ri = jnp.zeros((16,), jnp.int32)
                rm, ri = lax.fori_loop(0, nchunks // _UNROLL, chunk, (rm, ri))
                mv = bfly(rm, jnp.maximum)
                cand = jnp.where(
                    rm == mv, (ri * 16 + lane16).astype(jnp.float32), 1e9)
                lmin = bfly(cand, jnp.minimum)[0].astype(jnp.int32)
                m = mv[0]
                gidx = s * _TILE_PTS + lmin
                stagev = jnp.where(lane16 == j, m, stagev)
                stagev = jnp.where(lane16 == 8 + j, gidx.astype(jnp.float32), stagev)
            stage_pub[pl.ds(0, 16)] = stagev
            pltpu.sync_copy(stage_pub, shared.at[s])
            plsc.subcore_barrier()
            pltpu.sync_copy(shared.at[pl.ds(0, 16)], red_land)
            rot8 = (lane16 + 8) % 16
            m2v = jnp.full((16,), -jnp.inf, jnp.float32)
            for t in range(16):
                m2v = jnp.maximum(m2v, red_land[t, pl.ds(0, 16)])
            winv = jnp.full((16,), 1e9, jnp.float32)
            for t in range(16):
                row = red_land[t, pl.ds(0, 16)]
                sh = row.at[rot8].get(mode="promise_in_bounds")
                winv = jnp.minimum(winv, jnp.where(row == m2v, sh, 1e9))
            for j in range(_KC):
                publish_cent(j, winv[j].astype(jnp.int32))
            out_stage[pl.ds(0, 16)] = winv

            @pl.when(s == 0)
            def _():
                pltpu.sync_copy(out_stage, shared.at[16 + _KC + i])

            plsc.subcore_barrier()
            pltpu.sync_copy(shared.at[pl.ds(16, _KC)], cent_land)
            return 0

        lax.fori_loop(0, npoints, iter_body, 0)

        @pl.when(s == 0)
        def _():
            pltpu.sync_copy(shared.at[pl.ds(16 + _KC, npoints)], out_hbm.at[c])

    return k(xyzsc, far0sc)


def _fps_pallas(xyz, npoints):
    B, N, _ = xyz.shape
    far0 = jax.random.randint(jax.random.key(1), (B,), 0, N).astype(jnp.int32)
    if B <= _SC_B or N != _TILE_PTS * 16:
        return _fps_tc(xyz, far0, npoints)

    btc = B - _SC_B
    tc_out = _fps_tc(xyz[:btc], far0[:btc], npoints)

    xyzsc = jnp.transpose(
        xyz[btc:].reshape(2, _KC, 16, _TILE_PTS, 3), (0, 2, 4, 1, 3))
    far0sc = jnp.zeros((2, 128), jnp.int32).at[:, :_KC].set(
        far0[btc:].reshape(2, _KC))
    outsc = _fps_sc(xyzsc, far0sc, npoints)

    w = jnp.transpose(outsc[:, : npoints - 1, :_KC], (0, 2, 1)).reshape(
        _SC_B, npoints - 1).astype(jnp.int32)
    sc_out = jnp.concatenate([far0[btc:, None], w], axis=1)
    return jnp.concatenate([tc_out, sc_out], axis=0)


def kernel(xyz):
    return _fps_pallas(xyz, _NPOINTS)


# speedup vs baseline: 1.8020x; 1.8020x over previous
import functools

import jax
import jax.numpy as jnp
from jax import lax
from jax.experimental import pallas as pl
from jax.experimental.pallas import tpu as pltpu
from jax.experimental.pallas import tpu_sc as plsc

_NPOINTS = 512
_LANES = 128
_CH = 32

_SC_B = 2
_KC = 1
_TILE_PTS = 4096
_UNROLL = 4


def _amax_lanes(v):
    for s in (1, 2, 4, 8, 16, 32, 64):
        v = jnp.maximum(v, pltpu.roll(v, s, axis=1))
    return v


def _asum_lanes(v):
    for s in (1, 2, 4, 8, 16, 32, 64):
        v = v + pltpu.roll(v, s, axis=1)
    return v


def _fps_tc(xyz, far0, npoints):
    B, N, _ = xyz.shape
    rows = N // _LANES
    ch = min(_CH, rows)
    nchunks = rows // ch
    chunk_elems = ch * _LANES
    xyzt = jnp.transpose(xyz, (2, 0, 1)).reshape(3, B, rows, _LANES)
    xyz3 = jnp.transpose(xyz.reshape(B, rows, _LANES, 3), (0, 1, 3, 2))

    def body(far0_ref, xyzt_ref, xyz3_ref, out_ref, dist_ref):
        dist_ref[...] = jnp.full((B, rows, _LANES), 1e10, jnp.float32)
        lane3 = jax.lax.broadcasted_iota(jnp.int32, (3, _LANES), 1)
        b_iota = jax.lax.broadcasted_iota(jnp.int32, (1, B), 1)
        pos_iota = (
            jax.lax.broadcasted_iota(jnp.int32, (ch, _LANES), 0) * _LANES
            + jax.lax.broadcasted_iota(jnp.int32, (ch, _LANES), 1)
        )

        def gather_vec(b, f):
            g = xyz3_ref[b, pl.ds(f // _LANES, 1)].reshape(3, _LANES)
            return _asum_lanes(jnp.where(lane3 == f % _LANES, g, 0.0))

        def iter_body(i, carry):
            new_f = []
            new_c = []
            for b in range(B):
                cv = carry[b]
                cxb = jnp.broadcast_to(cv[0:1, :], (ch, _LANES))
                cyb = jnp.broadcast_to(cv[1:2, :], (ch, _LANES))
                czb = jnp.broadcast_to(cv[2:3, :], (ch, _LANES))
                rm = jnp.full((ch, _LANES), -1.0, jnp.float32)
                ri = jnp.zeros((ch, _LANES), jnp.int32)
                for k in range(nchunks):
                    sl = pl.ds(k * ch, ch)
                    x = xyzt_ref[0, b, sl, :]
                    y = xyzt_ref[1, b, sl, :]
                    z = xyzt_ref[2, b, sl, :]
                    dx = x - cxb
                    dy = y - cyb
                    dz = z - czb
                    d = dx * dx + dy * dy + dz * dz
                    nd = jnp.minimum(dist_ref[b, sl, :], d)
                    dist_ref[b, sl, :] = nd
                    gt = nd > rm
                    rm = jnp.where(gt, nd, rm)
                    ri = jnp.where(gt, k, ri)
                mvec = _amax_lanes(jnp.max(rm, axis=0, keepdims=True))
                f_sc = jnp.min(jnp.where(rm == mvec, ri * chunk_elems + pos_iota, N))
                new_f.append(f_sc)
                new_c.append(gather_vec(b, f_sc))
            rec = jnp.zeros((1, B), jnp.int32)
            for b in range(B):
                rec = jnp.where(b_iota == b, new_f[b], rec)
            out_ref[pl.ds(i + 1, 1), :] = rec
            return tuple(new_c)

        rec = jnp.zeros((1, B), jnp.int32)
        carry0 = []
        for b in range(B):
            f0 = far0_ref[b]
            rec = jnp.where(b_iota == b, f0, rec)
            carry0.append(gather_vec(b, f0))
        out_ref[pl.ds(0, 1), :] = rec

        jax.lax.fori_loop(0, npoints, iter_body, tuple(carry0))

    out = pl.pallas_call(
        body,
        grid=(),
        in_specs=[
            pl.BlockSpec(memory_space=pltpu.SMEM),
            pl.BlockSpec(memory_space=pltpu.VMEM),
            pl.BlockSpec(memory_space=pltpu.VMEM),
        ],
        out_specs=pl.BlockSpec(memory_space=pltpu.VMEM),
        out_shape=jax.ShapeDtypeStruct((npoints + 1, B), jnp.int32),
        scratch_shapes=[pltpu.VMEM((B, rows, _LANES), jnp.float32)],
    )(far0, xyzt, xyz3)
    return out[:npoints].T


def _fps_sc(xyzsc, far0sc, npoints):
    nchunks = _TILE_PTS // 16
    mesh = plsc.VectorSubcoreMesh(core_axis_name="c", subcore_axis_name="s")

    @functools.partial(
        pl.kernel,
        mesh=mesh,
        out_type=jax.ShapeDtypeStruct((2, npoints, 128), jnp.float32),
        scratch_types=[
            pltpu.VMEM((3, _KC, _TILE_PTS), jnp.float32),
            pltpu.VMEM((_KC, _TILE_PTS), jnp.float32),
            pltpu.VMEM((128,), jnp.int32),
            pltpu.VMEM((128,), jnp.float32),
            pltpu.VMEM((16, 128), jnp.float32),
            pltpu.VMEM((128,), jnp.float32),
            pltpu.VMEM((_KC, 128), jnp.float32),
            pltpu.VMEM((128,), jnp.float32),
            pltpu.VMEM_SHARED((16 + _KC + npoints, 128), jnp.float32),
        ],
    )
    def k(xyz_hbm, far0_hbm, out_hbm, xyz_sp, dist_sp, far_land, stage_pub,
          red_land, cent_stage, cent_land, out_stage, shared):
        c = lax.axis_index("c")
        s = lax.axis_index("s")
        lane16 = lax.iota(jnp.int32, 16)

        def bfly(v, op):
            for sh in (1, 2, 4, 8):
                p = v.at[(lane16 + sh) % 16].get(mode="promise_in_bounds")
                v = op(v, p)
            return v

        def lane_f32(v, l):
            return bfly(jnp.where(lane16 == l, v, -jnp.inf), jnp.maximum)[0]

        def lane_i32(v, l):
            return bfly(jnp.where(lane16 == l, v, 0), jnp.maximum)[0]

        pltpu.sync_copy(xyz_hbm.at[c, s], xyz_sp)
        pltpu.sync_copy(far0_hbm.at[c], far_land)
        for j in range(_KC):
            def init_d(ii, _, j=j):
                dist_sp[j, pl.ds(ii * 16, 16)] = jnp.full((16,), 1e10, jnp.float32)
                return 0
            lax.fori_loop(0, nchunks, init_d, 0)

        def publish_cent(j, f):
            owner = f // _TILE_PTS

            @pl.when(s == owner)
            def _():
                lp = f % _TILE_PTS
                base = (lp // 16) * 16
                sel = lane16 == lp - base
                cx = bfly(jnp.where(sel, xyz_sp[0, j, pl.ds(base, 16)],
                                    -jnp.inf), jnp.maximum)[0]
                cy = bfly(jnp.where(sel, xyz_sp[1, j, pl.ds(base, 16)],
                                    -jnp.inf), jnp.maximum)[0]
                cz = bfly(jnp.where(sel, xyz_sp[2, j, pl.ds(base, 16)],
                                    -jnp.inf), jnp.maximum)[0]
                crow = jnp.where(
                    lane16 == 0, cx,
                    jnp.where(lane16 == 1, cy,
                              jnp.where(lane16 == 2, cz, 0.0)))
                cent_stage[pl.ds(0, 16)] = crow
                pltpu.sync_copy(cent_stage, shared.at[16 + j])

        farv = far_land[pl.ds(0, 16)]
        for j in range(_KC):
            publish_cent(j, lane_i32(farv, j))
        plsc.subcore_barrier()
        pltpu.sync_copy(shared.at[pl.ds(16, _KC)], cent_land)

        def iter_body(i, _):
            stagev = jnp.zeros((16,), jnp.float32)
            for j in range(_KC):
                crow = cent_land[j, pl.ds(0, 16)]
                cx = crow[0]
                cy = crow[1]
                cz = crow[2]

                def chunk(ii, carry, j=j, cx=cx, cy=cy, cz=cz):
                    rm, ri = carry
                    for u in range(_UNROLL):
                        kk = ii * _UNROLL + u
                        sl = pl.ds(kk * 16, 16)
                        x = xyz_sp[0, j, sl]
                        y = xyz_sp[1, j, sl]
                        z = xyz_sp[2, j, sl]
                        dx = x - cx
                        dy = y - cy
                        dz = z - cz
                        d = dx * dx + dy * dy + dz * dz
                        nd = jnp.minimum(dist_sp[j, sl], d)
                        dist_sp[j, sl] = nd
                        gt = nd > rm
                        rm = jnp.where(gt, nd, rm)
                        ri = jnp.where(gt, jnp.full((16,), kk, jnp.int32), ri)
                    return rm, ri

                rm = jnp.full((16,), -1.0, jnp.float32)
                ri = jnp.zeros((16,), jnp.int32)
                rm, ri = lax.fori_loop(0, nchunks // _UNROLL, chunk, (rm, ri))
                mv = bfly(rm, jnp.maximum)
                cand = jnp.where(
                    rm == mv, (ri * 16 + lane16).astype(jnp.float32), 1e9)
                lmin = bfly(cand, jnp.minimum)[0].astype(jnp.int32)
                m = mv[0]
                gidx = s * _TILE_PTS + lmin
                stagev = jnp.where(lane16 == j, m, stagev)
                stagev = jnp.where(lane16 == 8 + j, gidx.astype(jnp.float32), stagev)
            stage_pub[pl.ds(0, 16)] = stagev
            pltpu.sync_copy(stage_pub, shared.at[s])
            plsc.subcore_barrier()
            pltpu.sync_copy(shared.at[pl.ds(0, 16), pl.ds(0, 16)],
                            red_land.at[:, pl.ds(0, 16)])
            rot8 = (lane16 + 8) % 16
            m2v = jnp.full((16,), -jnp.inf, jnp.float32)
            for t in range(16):
                m2v = jnp.maximum(m2v, red_land[t, pl.ds(0, 16)])
            winv = jnp.full((16,), 1e9, jnp.float32)
            for t in range(16):
                row = red_land[t, pl.ds(0, 16)]
                sh = row.at[rot8].get(mode="promise_in_bounds")
                winv = jnp.minimum(winv, jnp.where(row == m2v, sh, 1e9))
            for j in range(_KC):
                publish_cent(j, winv[j].astype(jnp.int32))
            out_stage[pl.ds(0, 16)] = winv

            @pl.when(s == 0)
            def _():
                pltpu.sync_copy(out_stage, shared.at[16 + _KC + i])

            plsc.subcore_barrier()
            pltpu.sync_copy(shared.at[pl.ds(16, _KC)], cent_land)
            return 0

        lax.fori_loop(0, npoints, iter_body, 0)

        @pl.when(s == 0)
        def _():
            pltpu.sync_copy(shared.at[pl.ds(16 + _KC, npoints)], out_hbm.at[c])

    return k(xyzsc, far0sc)


def _fps_pallas(xyz, npoints):
    B, N, _ = xyz.shape
    far0 = jax.random.randint(jax.random.key(1), (B,), 0, N).astype(jnp.int32)
    if B <= _SC_B or N != _TILE_PTS * 16:
        return _fps_tc(xyz, far0, npoints)

    btc = B - _SC_B
    tc_out = _fps_tc(xyz[:btc], far0[:btc], npoints)

    xyzsc = jnp.transpose(
        xyz[btc:].reshape(2, _KC, 16, _TILE_PTS, 3), (0, 2, 4, 1, 3))
    far0sc = jnp.zeros((2, 128), jnp.int32).at[:, :_KC].set(
        far0[btc:].reshape(2, _KC))
    outsc = _fps_sc(xyzsc, far0sc, npoints)

    w = jnp.transpose(outsc[:, : npoints - 1, :_KC], (0, 2, 1)).reshape(
        _SC_B, npoints - 1).astype(jnp.int32)
    sc_out = jnp.concatenate([far0[btc:, None], w], axis=1)
    return jnp.concatenate([tc_out, sc_out], axis=0)


def kernel(xyz):
    return _fps_pallas(xyz, _NPOINTS)
